# Initial kernel scaffold; baseline (speedup 1.0000x reference)
import jax
import jax.numpy as jnp
from jax.experimental import pallas as pl


def kernel(x, edge_index, edge_attr, batch, params):
    raise NotImplementedError("write your pallas kernel here")


# speedup vs baseline: 4.3950x; 4.3950x over previous
import functools

import jax
import jax.numpy as jnp
from jax import lax
from jax.experimental import pallas as pl
from jax.experimental.pallas import tpu as pltpu
from jax.experimental.pallas import tpu_sc as plsc

F32 = jnp.float32
I32 = jnp.int32

_NC = 2
_NS = 16
_NW = _NC * _NS
_LW = 128
_CH = 16

_TE = 4000
_TN = 2000


def _sp(x):
    return jnp.maximum(x, 0.0) + jnp.log(1.0 + jnp.exp(-jnp.abs(x)))


def _full(a):
    return pl.BlockSpec(a.shape, lambda i: (0,) * a.ndim)


def _row_spec(t, w):
    return pl.BlockSpec((t, w), lambda i: (i, 0))


def _edge_init(edge_attr, w1t, b1, w2t, b2):
    E = edge_attr.shape[0]
    G = E // _TE

    def body(a_ref, w1_ref, b1_ref, w2_ref, b2_ref, o_ref):
        a = a_ref[...]
        d = jnp.sqrt(jnp.sum(a * a, axis=1, keepdims=True))
        cen = lax.broadcasted_iota(I32, (1, 100), 1).astype(F32) * (5.0 / 99.0)
        r = jnp.exp(-4.0 * (d - cen) ** 2)
        h = _sp(jnp.dot(r, w1_ref[...]) + b1_ref[...])
        o_ref[...] = _sp(jnp.dot(h, w2_ref[...]) + b2_ref[...])

    return pl.pallas_call(
        body, grid=(G,),
        in_specs=[_row_spec(_TE, 3), _full(w1t), _full(b1), _full(w2t), _full(b2)],
        out_specs=_row_spec(_TE, 32),
        out_shape=jax.ShapeDtypeStruct((E, 32), F32),
    )(edge_attr, w1t, b1, w2t, b2)


def _node_init(x, wet, be, w1t, b1, w2t, b2):
    N = x.shape[0]
    G = N // _TN

    def body(x_ref, we_ref, be_ref, w1_ref, b1_ref, w2_ref, b2_ref, o_ref):
        v = jnp.dot(x_ref[...], we_ref[...]) + be_ref[...]
        h = _sp(jnp.dot(v, w1_ref[...]) + b1_ref[...])
        o_ref[...] = _sp(jnp.dot(h, w2_ref[...]) + b2_ref[...])

    return pl.pallas_call(
        body, grid=(G,),
        in_specs=[_row_spec(_TN, x.shape[1])] + [_full(a) for a in (wet, be, w1t, b1, w2t, b2)],
        out_specs=_row_spec(_TN, 32),
        out_shape=jax.ShapeDtypeStruct((N, 32), F32),
    )(x, wet, be, w1t, b1, w2t, b2)


def _node_ff(v, w1t, b1, w2t, b2):
    N = v.shape[0]
    G = N // _TN

    def body(v_ref, w1_ref, b1_ref, w2_ref, b2_ref, o_ref):
        h = _sp(jnp.dot(v_ref[...], w1_ref[...]) + b1_ref[...])
        o_ref[...] = _sp(jnp.dot(h, w2_ref[...]) + b2_ref[...])

    return pl.pallas_call(
        body, grid=(G,),
        in_specs=[_row_spec(_TN, 32)] + [_full(a) for a in (w1t, b1, w2t, b2)],
        out_specs=_row_spec(_TN, 32),
        out_shape=jax.ShapeDtypeStruct((N, 32), F32),
    )(v, w1t, b1, w2t, b2)


def _edge_conv(eres, vs, vd, src_col, lo, hi, u, ffw, cw, epad):
    E = eres.shape[0]
    G = E // _TE
    has_ff = ffw is not None

    def body(*refs):
        e_ref, vs_ref, vd_ref, s_ref, lo_ref, hi_ref, u_ref = refs[:7]
        k = 7
        if has_ff:
            fw1, fb1, fw2, fb2 = (r[...] for r in refs[k:k + 4])
            k += 4
        w1, bb1, w2, bb2, w3, bb3 = (r[...] for r in refs[k:k + 6])
        ep_ref, en_ref = refs[k + 6:k + 8]
        er = e_ref[...]
        if has_ff:
            eff = _sp(jnp.dot(_sp(jnp.dot(er, fw1) + fb1), fw2) + fb2)
        else:
            eff = er
        s = s_ref[...]
        oh = ((s >= lo_ref[...]) & (s < hi_ref[...])).astype(F32)
        ub = jnp.dot(oh, u_ref[...])
        cc = jnp.concatenate([vs_ref[...], vd_ref[...], ub, eff], axis=1)
        h = _sp(jnp.dot(cc, w1) + bb1)
        h = _sp(jnp.dot(h, w2) + bb2)
        ep = _sp(jnp.dot(h, w3) + bb3)
        ep_ref[...] = ep
        en_ref[...] = ep + er

    ins = [eres, vs, vd, src_col, lo, hi, u] + (list(ffw) if has_ff else []) + list(cw)
    in_specs = ([_row_spec(_TE, 32)] * 3 + [_row_spec(_TE, 1)]
                + [_full(a) for a in ins[4:]])
    return pl.pallas_call(
        body, grid=(G,),
        in_specs=in_specs,
        out_specs=[_row_spec(_TE, 32)] * 2,
        out_shape=[jax.ShapeDtypeStruct((epad, 32), F32),
                   jax.ShapeDtypeStruct((E, 32), F32)],
    )(*ins)


def _node_conv(vres, vff, sums, c0, c1, u, lo, hi, cw):
    N = vres.shape[0]
    G = N // _TN

    def body(vr_ref, vf_ref, sm_ref, c0_ref, c1_ref, u_ref, lo_ref, hi_ref,
             w1_ref, b1_ref, w2_ref, b2_ref, w3_ref, b3_ref,
             vn_ref, ue_ref, uv_ref, ue_acc, uv_acc):
        i = pl.program_id(0)
        nid = i * _TN + lax.broadcasted_iota(I32, (_TN, 1), 0)
        oh = ((nid >= lo_ref[...]) & (nid < hi_ref[...])).astype(F32)
        cnt = jnp.maximum(c0_ref[...][:, :1] + c1_ref[...][:, :1], 1.0)
        etv = sm_ref[...] / cnt
        ub = jnp.dot(oh, u_ref[...])
        cc = jnp.concatenate([vf_ref[...], etv, ub], axis=1)
        h = _sp(jnp.dot(cc, w1_ref[...]) + b1_ref[...])
        h = _sp(jnp.dot(h, w2_ref[...]) + b2_ref[...])
        vp = _sp(jnp.dot(h, w3_ref[...]) + b3_ref[...])
        vn_ref[...] = vp + vr_ref[...]

        @pl.when(i == 0)
        def _():
            ue_acc[...] = jnp.zeros_like(ue_acc)
            uv_acc[...] = jnp.zeros_like(uv_acc)

        dn = (((0,), (0,)), ((), ()))
        ue_acc[...] += lax.dot_general(oh, etv, dn)
        uv_acc[...] += lax.dot_general(oh, vp, dn)

        @pl.when(i == G - 1)
        def _():
            ue_ref[...] = ue_acc[...]
            uv_ref[...] = uv_acc[...]

    ins = [vres, vff, sums, c0, c1, u, lo, hi] + list(cw)
    in_specs = ([_row_spec(_TN, 32)] * 3 + [_row_spec(_TN, 16)] * 2
                + [_full(a) for a in ins[5:]])
    return pl.pallas_call(
        body, grid=(G,),
        in_specs=in_specs,
        out_specs=[_row_spec(_TN, 32), _full(jnp.zeros((64, 32))), _full(jnp.zeros((64, 32)))],
        out_shape=[jax.ShapeDtypeStruct((N, 32), F32),
                   jax.ShapeDtypeStruct((64, 32), F32),
                   jax.ShapeDtypeStruct((64, 32), F32)],
        scratch_shapes=[pltpu.VMEM((64, 32), F32), pltpu.VMEM((64, 32), F32)],
    )(*ins)


def _linear16(x, wt, b, t):
    M = x.shape[0]
    G = M // t

    def body(x_ref, w_ref, b_ref, o_ref):
        o_ref[...] = jnp.dot(x_ref[...], w_ref[...]) + b_ref[...]

    return pl.pallas_call(
        body, grid=(G,),
        in_specs=[_row_spec(t, 32), _full(wt), _full(b)],
        out_specs=_row_spec(t, 16),
        out_shape=jax.ShapeDtypeStruct((M, 16), F32),
    )(x, wt, b)


def _s2s_pass(xx, ids_col, q, lo, hi, t):
    M = xx.shape[0]
    G = M // t
    node_mode = ids_col is None

    def body(*refs):
        if node_mode:
            x_ref, q_ref, lo_ref, hi_ref = refs[:4]
            k = 4
        else:
            x_ref, id_ref, q_ref, lo_ref, hi_ref = refs[:5]
            k = 5
        m_ref, s_ref, r_ref, m_acc, s_acc, r_acc = refs[k:k + 6]
        i = pl.program_id(0)
        if node_mode:
            ids = i * t + lax.broadcasted_iota(I32, (t, 1), 0)
        else:
            ids = id_ref[...]
        oh = ((ids >= lo_ref[...]) & (ids < hi_ref[...])).astype(F32)
        xv = x_ref[...]
        qe = jnp.dot(oh, q_ref[...])
        e = jnp.sum(xv * qe, axis=1, keepdims=True)
        masked = oh * e - (1.0 - oh) * 1e30
        mt = jnp.max(masked, axis=0, keepdims=True)

        @pl.when(i == 0)
        def _():
            m_acc[...] = jnp.full_like(m_acc, -1e30)
            s_acc[...] = jnp.zeros_like(s_acc)
            r_acc[...] = jnp.zeros_like(r_acc)

        m_old = m_acc[...]
        m_new = jnp.maximum(m_old, mt)
        scale = jnp.exp(m_old - m_new)
        m_e = jnp.sum(oh * m_new, axis=1, keepdims=True)
        p = jnp.exp(e - m_e)
        w = oh * p
        s_acc[...] = s_acc[...] * scale + jnp.sum(w, axis=0, keepdims=True)
        r_acc[...] = r_acc[...] * scale + lax.dot_general(
            xv, w, (((0,), (0,)), ((), ())))
        m_acc[...] = m_new

        @pl.when(i == G - 1)
        def _():
            m_ref[...] = m_acc[...]
            s_ref[...] = s_acc[...]
            r_ref[...] = r_acc[...]

    ins = [xx] + ([] if node_mode else [ids_col]) + [q, lo, hi]
    in_specs = [_row_spec(t, 16)] + ([] if node_mode else [_row_spec(t, 1)]) \
        + [_full(a) for a in (q, lo, hi)]
    return pl.pallas_call(
        body, grid=(G,),
        in_specs=in_specs,
        out_specs=[_full(jnp.zeros((1, 64))), _full(jnp.zeros((1, 64))),
                   _full(jnp.zeros((16, 64)))],
        out_shape=[jax.ShapeDtypeStruct((1, 64), F32),
                   jax.ShapeDtypeStruct((1, 64), F32),
                   jax.ShapeDtypeStruct((16, 64), F32)],
        scratch_shapes=[pltpu.VMEM((1, 64), F32), pltpu.VMEM((1, 64), F32),
                        pltpu.VMEM((16, 64), F32)],
    )(*ins)


def _sc_mesh():
    return plsc.VectorSubcoreMesh(core_axis_name="c", subcore_axis_name="s")


def _sc_gather2(tab, srcp, dstp):
    npad = tab.shape[0]
    re_rows = srcp.shape[0]
    epad = re_rows * _LW
    rpw = re_rows // _NW
    nb = rpw // _CH
    eb = _CH * _LW

    def body(tab_hbm, si_hbm, di_hbm, vs_hbm, vd_hbm, idx_v, rows_v, sem):
        c = lax.axis_index("c")
        s = lax.axis_index("s")
        wid = s * _NC + c

        def block(j2, idx_hbm, out_hbm):
            rb = wid * rpw + j2 * _CH
            pltpu.sync_copy(idx_hbm.at[pl.ds(rb, _CH)], idx_v)
            hs = [pltpu.async_copy(tab_hbm.at[idx_v.at[j]],
                                   rows_v.at[pl.ds(j * _LW, _LW)], sem)
                  for j in range(_CH)]
            for h in hs:
                h.wait()
            pltpu.sync_copy(rows_v, out_hbm.at[pl.ds(rb * _LW, eb)])

        def loop_s(j2, carry):
            block(j2, si_hbm, vs_hbm)
            return carry

        def loop_d(j2, carry):
            block(j2, di_hbm, vd_hbm)
            return carry

        lax.fori_loop(0, nb, loop_s, 0)
        lax.fori_loop(0, nb, loop_d, 0)

    f = pl.kernel(
        body,
        out_type=[jax.ShapeDtypeStruct((epad, 32), F32),
                  jax.ShapeDtypeStruct((epad, 32), F32)],
        mesh=_sc_mesh(),
        compiler_params=pltpu.CompilerParams(use_tc_tiling_on_sc=False),
        scratch_types=[pltpu.VMEM((_CH, _LW), I32),
                       pltpu.VMEM((eb, 32), F32),
                       pltpu.SemaphoreType.DMA],
    )
    return f(tab, srcp, dstp)


def _sc_scatter(ep, dstp, npad):
    ch = 8
    re_rows = dstp.shape[0]
    rpt = re_rows // _NS
    nb = rpt // ch
    eb = ch * _LW
    rt = npad // _NS
    zch = rt // 8

    def body(ep_hbm, di_hbm, out_hbm, acc_sh, idx_v, dat_v, sem):
        c = lax.axis_index("c")
        s = lax.axis_index("s")

        def zb(i, carry):
            dat_v[i, :] = jnp.zeros((16,), F32)
            return carry

        lax.fori_loop(0, zch, zb, 0)

        def zc(k, carry):
            pltpu.sync_copy(dat_v.at[pl.ds(0, zch)],
                            acc_sh.at[pl.ds(s * rt + k * zch, zch)])
            return carry

        lax.fori_loop(0, 8, zc, 0)
        plsc.subcore_barrier()

        def loop(j2, carry):
            rb = s * rpt + j2 * ch
            pltpu.sync_copy(di_hbm.at[pl.ds(rb, ch)], idx_v)
            pltpu.sync_copy(ep_hbm.at[pl.ds(rb * _LW, eb), pl.ds(c * 16, 16)],
                            dat_v)
            hs = [pltpu.async_copy(dat_v.at[pl.ds(j * _LW, _LW)],
                                   acc_sh.at[idx_v.at[j]], sem, add=True)
                  for j in range(ch)]
            for h in hs:
                h.wait()
            return carry

        lax.fori_loop(0, nb, loop, 0)
        plsc.subcore_barrier()

        def wb(k, carry):
            r0 = s * rt + k * zch
            pltpu.sync_copy(acc_sh.at[pl.ds(r0, zch)], dat_v.at[pl.ds(0, zch)])
            pltpu.sync_copy(dat_v.at[pl.ds(0, zch)],
                            out_hbm.at[pl.ds(r0, zch), pl.ds(c * 16, 16)])
            return carry

        lax.fori_loop(0, 8, wb, 0)

    f = pl.kernel(
        body,
        out_type=jax.ShapeDtypeStruct((npad, 32), F32),
        mesh=_sc_mesh(),
        compiler_params=pltpu.CompilerParams(use_tc_tiling_on_sc=False),
        scratch_types=[pltpu.VMEM_SHARED((npad, 16), F32),
                       pltpu.VMEM((ch, _LW), I32),
                       pltpu.VMEM((eb, 16), F32),
                       pltpu.SemaphoreType.DMA],
    )
    return f(ep, dstp)


def _sc_counts(dstp, npad):
    re_rows = dstp.shape[0]
    rpsc = re_rows // _NC
    rpt = rpsc // _NS
    nb = rpt // _CH
    rt = npad // _NS

    def body(di_hbm, out_hbm, acc_sh, idx_v, one_v, sem):
        c = lax.axis_index("c")
        s = lax.axis_index("s")

        def zb(i, carry):
            one_v[i, :] = jnp.zeros((16,), F32)
            return carry

        lax.fori_loop(0, _LW, zb, 0)

        def zc(k, carry):
            pltpu.sync_copy(one_v, acc_sh.at[pl.ds(s * rt + k * _LW, _LW)])
            return carry

        lax.fori_loop(0, rt // _LW, zc, 0)
        plsc.subcore_barrier()

        def ob(i, carry):
            one_v[i, :] = jnp.ones((16,), F32)
            return carry

        lax.fori_loop(0, _LW, ob, 0)

        def loop(j2, carry):
            rb = c * rpsc + s * rpt + j2 * _CH
            pltpu.sync_copy(di_hbm.at[pl.ds(rb, _CH)], idx_v)
            hs = [pltpu.async_copy(one_v, acc_sh.at[idx_v.at[j]], sem, add=True)
                  for j in range(_CH)]
            for h in hs:
                h.wait()
            return carry

        lax.fori_loop(0, nb, loop, 0)
        plsc.subcore_barrier()

        def wb(k, carry):
            r0 = s * rt + k * _LW
            pltpu.sync_copy(acc_sh.at[pl.ds(r0, _LW)], one_v)
            pltpu.sync_copy(one_v, out_hbm.at[c, pl.ds(r0, _LW)])
            return carry

        lax.fori_loop(0, rt // _LW, wb, 0)

    f = pl.kernel(
        body,
        out_type=jax.ShapeDtypeStruct((_NC, npad, 16), F32),
        mesh=_sc_mesh(),
        compiler_params=pltpu.CompilerParams(use_tc_tiling_on_sc=False),
        scratch_types=[pltpu.VMEM_SHARED((npad, 16), F32),
                       pltpu.VMEM((_CH, _LW), I32),
                       pltpu.VMEM((_LW, 16), F32),
                       pltpu.SemaphoreType.DMA],
    )
    return f(dstp)


def _lin_j(x, wb):
    w, b = wb
    return jnp.dot(x, w.T) + b


def _ff_j(x, pp):
    return jax.nn.softplus(_lin_j(jax.nn.softplus(_lin_j(x, pp[0])), pp[1]))


def _set2set(xx, ids_col, lp, lo, hi, t):
    w_ih, w_hh, b_ih, b_hh = lp
    q_star = jnp.zeros((64, 32), F32)
    h = jnp.zeros((64, 16), F32)
    cst = jnp.zeros((64, 16), F32)
    for _ in range(3):
        gates = jnp.dot(q_star, w_ih.T) + b_ih + jnp.dot(h, w_hh.T) + b_hh
        ig, fg, gg, og = jnp.split(gates, 4, axis=1)
        ig = jax.nn.sigmoid(ig)
        fg = jax.nn.sigmoid(fg)
        gg = jnp.tanh(gg)
        og = jax.nn.sigmoid(og)
        cst = fg * cst + ig * gg
        h = og * jnp.tanh(cst)
        q = h
        _, sexp, r = _s2s_pass(xx, ids_col, q, lo, hi, t)
        rg = (r / (sexp + 1e-16)).T
        q_star = jnp.concatenate([q, rg], axis=1)
    return q_star


def _t2(wb):
    w, b = wb
    return w.T, b.reshape(1, -1)


def kernel(x, edge_index, edge_attr, batch, params):
    p = params
    N = x.shape[0]
    E = edge_index.shape[1]
    unit = _NW * _CH * _LW
    epad = ((E + unit - 1) // unit) * unit
    npad = (N // 512 + 1) * 512
    trash = npad - 1
    re_rows = epad // _LW

    src = edge_index[0].astype(I32)
    dst = edge_index[1].astype(I32)
    padv = jnp.full((epad - E,), trash, I32)
    srcp = jnp.concatenate([src, padv]).reshape(re_rows, _LW)
    dstp = jnp.concatenate([dst, padv]).reshape(re_rows, _LW)
    src_col = src.reshape(E, 1)
    dst_col = dst.reshape(E, 1)

    starts = jnp.searchsorted(batch, jnp.arange(65, dtype=I32)).astype(I32)
    lo = starts[:64].reshape(1, 64)
    hi = starts[1:].reshape(1, 64)
    bcnt = jnp.maximum((starts[1:] - starts[:64]).astype(F32), 1.0).reshape(64, 1)

    e0 = _edge_init(edge_attr, *_t2(p['ffe0'][0]), *_t2(p['ffe0'][1]))
    v0 = _node_init(x, *_t2(p['embedding']), *_t2(p['ffv0'][0]), *_t2(p['ffv0'][1]))
    u0 = _ff_j(jnp.zeros((64, 2), F32), p['ffu0'])

    cnts = _sc_counts(dstp, npad)
    c0 = cnts[0]
    c1 = cnts[1]

    vres, eres, ures = v0, e0, u0
    for li, mk in enumerate(('meg1', 'meg2', 'meg3')):
        mp = p[mk]
        if li == 0:
            vff, uin, ew = vres, ures, None
        else:
            fv, fe, fu = p['ffv%d' % li], p['ffe%d' % li], p['ffu%d' % li]
            vff = _node_ff(vres, *_t2(fv[0]), *_t2(fv[1]))
            ew = (*_t2(fe[0]), *_t2(fe[1]))
            uin = _ff_j(ures, fu)
        tab = jnp.pad(vff, ((0, npad - N), (0, 0)))
        vs, vd = _sc_gather2(tab, srcp, dstp)
        cw_e = (*_t2(mp['e1']), *_t2(mp['e2']), *_t2(mp['e3']))
        ep, enext = _edge_conv(eres, vs, vd, src_col, lo, hi, uin, ew, cw_e, epad)
        sums = _sc_scatter(ep, dstp, npad)
        cw_v = (*_t2(mp['v1']), *_t2(mp['v2']), *_t2(mp['v3']))
        vnext, ue_s, uv_s = _node_conv(vres, vff, sums, c0, c1, uin, lo, hi, cw_v)
        ue = ue_s / bcnt
        uv = uv_s / bcnt
        uc = jnp.concatenate([ue, uv, uin], axis=1)
        upd = jax.nn.softplus(_lin_j(uc, mp['u1']))
        upd = jax.nn.softplus(_lin_j(upd, mp['u2']))
        upd = jax.nn.softplus(_lin_j(upd, mp['u3']))
        vres, eres, ures = vnext, enext, upd + ures

    xv = _linear16(vres, *_t2(p['node_linear']), _TN)
    xe = _linear16(eres, *_t2(p['edge_linear']), _TE)
    node_vec = _set2set(xv, None, p['lstm_node'], lo, hi, _TN)
    edge_vec = _set2set(xe, dst_col, p['lstm_edge'], lo, hi, _TE)
    fin = jnp.concatenate([node_vec, edge_vec, ures], axis=1)
    out = jax.nn.softplus(_lin_j(fin, p['fc1']))
    out = jax.nn.softplus(_lin_j(out, p['fc2']))
    return _lin_j(out, p['fc3'])


# speedup vs baseline: 4.4583x; 1.0144x over previous
import functools

import jax
import jax.numpy as jnp
from jax import lax
from jax.experimental import pallas as pl
from jax.experimental.pallas import tpu as pltpu
from jax.experimental.pallas import tpu_sc as plsc

F32 = jnp.float32
I32 = jnp.int32

_NC = 2
_NS = 16
_NW = _NC * _NS
_LW = 128
_CH = 16

_TE = 4000
_TN = 2000


def _sp(x):
    return jnp.maximum(x, 0.0) + jnp.log(1.0 + jnp.exp(-jnp.abs(x)))


def _full(a):
    return pl.BlockSpec(a.shape, lambda i: (0,) * a.ndim)


def _row_spec(t, w):
    return pl.BlockSpec((t, w), lambda i: (i, 0))


def _edge_init(edge_attr, w1t, b1, w2t, b2):
    E = edge_attr.shape[0]
    G = E // _TE

    def body(a_ref, w1_ref, b1_ref, w2_ref, b2_ref, o_ref):
        a = a_ref[...]
        d = jnp.sqrt(jnp.sum(a * a, axis=1, keepdims=True))
        cen = lax.broadcasted_iota(I32, (1, 100), 1).astype(F32) * (5.0 / 99.0)
        r = jnp.exp(-4.0 * (d - cen) ** 2)
        h = _sp(jnp.dot(r, w1_ref[...]) + b1_ref[...])
        o_ref[...] = _sp(jnp.dot(h, w2_ref[...]) + b2_ref[...])

    return pl.pallas_call(
        body, grid=(G,),
        in_specs=[_row_spec(_TE, 3), _full(w1t), _full(b1), _full(w2t), _full(b2)],
        out_specs=_row_spec(_TE, 32),
        out_shape=jax.ShapeDtypeStruct((E, 32), F32),
    )(edge_attr, w1t, b1, w2t, b2)


def _node_init(x, wet, be, w1t, b1, w2t, b2):
    N = x.shape[0]
    G = N // _TN

    def body(x_ref, we_ref, be_ref, w1_ref, b1_ref, w2_ref, b2_ref, o_ref):
        v = jnp.dot(x_ref[...], we_ref[...]) + be_ref[...]
        h = _sp(jnp.dot(v, w1_ref[...]) + b1_ref[...])
        o_ref[...] = _sp(jnp.dot(h, w2_ref[...]) + b2_ref[...])

    return pl.pallas_call(
        body, grid=(G,),
        in_specs=[_row_spec(_TN, x.shape[1])] + [_full(a) for a in (wet, be, w1t, b1, w2t, b2)],
        out_specs=_row_spec(_TN, 32),
        out_shape=jax.ShapeDtypeStruct((N, 32), F32),
    )(x, wet, be, w1t, b1, w2t, b2)


def _node_ff(v, w1t, b1, w2t, b2):
    N = v.shape[0]
    G = N // _TN

    def body(v_ref, w1_ref, b1_ref, w2_ref, b2_ref, o_ref):
        h = _sp(jnp.dot(v_ref[...], w1_ref[...]) + b1_ref[...])
        o_ref[...] = _sp(jnp.dot(h, w2_ref[...]) + b2_ref[...])

    return pl.pallas_call(
        body, grid=(G,),
        in_specs=[_row_spec(_TN, 32)] + [_full(a) for a in (w1t, b1, w2t, b2)],
        out_specs=_row_spec(_TN, 32),
        out_shape=jax.ShapeDtypeStruct((N, 32), F32),
    )(v, w1t, b1, w2t, b2)


def _edge_conv(eres, vs, vd, src_col, lo, hi, u, ffw, cw, epad):
    E = eres.shape[0]
    G = E // _TE
    has_ff = ffw is not None

    def body(*refs):
        e_ref, vs_ref, vd_ref, s_ref, lo_ref, hi_ref, u_ref = refs[:7]
        k = 7
        if has_ff:
            fw1, fb1, fw2, fb2 = (r[...] for r in refs[k:k + 4])
            k += 4
        w1, bb1, w2, bb2, w3, bb3 = (r[...] for r in refs[k:k + 6])
        ep_ref, en_ref = refs[k + 6:k + 8]
        er = e_ref[...]
        if has_ff:
            eff = _sp(jnp.dot(_sp(jnp.dot(er, fw1) + fb1), fw2) + fb2)
        else:
            eff = er
        s = s_ref[...]
        oh = ((s >= lo_ref[...]) & (s < hi_ref[...])).astype(F32)
        ub = jnp.dot(oh, u_ref[...])
        cc = jnp.concatenate([vs_ref[...], vd_ref[...], ub, eff], axis=1)
        h = _sp(jnp.dot(cc, w1) + bb1)
        h = _sp(jnp.dot(h, w2) + bb2)
        ep = _sp(jnp.dot(h, w3) + bb3)
        ep_ref[...] = ep
        en_ref[...] = ep + er

    ins = [eres, vs, vd, src_col, lo, hi, u] + (list(ffw) if has_ff else []) + list(cw)
    in_specs = ([_row_spec(_TE, 32)] * 3 + [_row_spec(_TE, 1)]
                + [_full(a) for a in ins[4:]])
    return pl.pallas_call(
        body, grid=(G,),
        in_specs=in_specs,
        out_specs=[_row_spec(_TE, 32)] * 2,
        out_shape=[jax.ShapeDtypeStruct((epad, 32), F32),
                   jax.ShapeDtypeStruct((E, 32), F32)],
    )(*ins)


def _node_conv(vres, vff, sums, c0, c1, u, lo, hi, cw):
    N = vres.shape[0]
    G = N // _TN

    def body(vr_ref, vf_ref, sm_ref, c0_ref, c1_ref, u_ref, lo_ref, hi_ref,
             w1_ref, b1_ref, w2_ref, b2_ref, w3_ref, b3_ref,
             vn_ref, ue_ref, uv_ref, ue_acc, uv_acc):
        i = pl.program_id(0)
        nid = i * _TN + lax.broadcasted_iota(I32, (_TN, 1), 0)
        oh = ((nid >= lo_ref[...]) & (nid < hi_ref[...])).astype(F32)
        cnt = jnp.maximum(c0_ref[...][:, :1] + c1_ref[...][:, :1], 1.0)
        etv = sm_ref[...] / cnt
        ub = jnp.dot(oh, u_ref[...])
        cc = jnp.concatenate([vf_ref[...], etv, ub], axis=1)
        h = _sp(jnp.dot(cc, w1_ref[...]) + b1_ref[...])
        h = _sp(jnp.dot(h, w2_ref[...]) + b2_ref[...])
        vp = _sp(jnp.dot(h, w3_ref[...]) + b3_ref[...])
        vn_ref[...] = vp + vr_ref[...]

        @pl.when(i == 0)
        def _():
            ue_acc[...] = jnp.zeros_like(ue_acc)
            uv_acc[...] = jnp.zeros_like(uv_acc)

        dn = (((0,), (0,)), ((), ()))
        ue_acc[...] += lax.dot_general(oh, etv, dn)
        uv_acc[...] += lax.dot_general(oh, vp, dn)

        @pl.when(i == G - 1)
        def _():
            ue_ref[...] = ue_acc[...]
            uv_ref[...] = uv_acc[...]

    ins = [vres, vff, sums, c0, c1, u, lo, hi] + list(cw)
    in_specs = ([_row_spec(_TN, 32)] * 3 + [_row_spec(_TN, 16)] * 2
                + [_full(a) for a in ins[5:]])
    return pl.pallas_call(
        body, grid=(G,),
        in_specs=in_specs,
        out_specs=[_row_spec(_TN, 32), _full(jnp.zeros((64, 32))), _full(jnp.zeros((64, 32)))],
        out_shape=[jax.ShapeDtypeStruct((N, 32), F32),
                   jax.ShapeDtypeStruct((64, 32), F32),
                   jax.ShapeDtypeStruct((64, 32), F32)],
        scratch_shapes=[pltpu.VMEM((64, 32), F32), pltpu.VMEM((64, 32), F32)],
    )(*ins)


def _linear16(x, wt, b, t):
    M = x.shape[0]
    G = M // t

    def body(x_ref, w_ref, b_ref, o_ref):
        o_ref[...] = jnp.dot(x_ref[...], w_ref[...]) + b_ref[...]

    return pl.pallas_call(
        body, grid=(G,),
        in_specs=[_row_spec(t, 32), _full(wt), _full(b)],
        out_specs=_row_spec(t, 16),
        out_shape=jax.ShapeDtypeStruct((M, 16), F32),
    )(x, wt, b)


def _s2s_pass(xx, ids_col, q, lo, hi, t):
    M = xx.shape[0]
    G = M // t
    node_mode = ids_col is None

    def body(*refs):
        if node_mode:
            x_ref, q_ref, lo_ref, hi_ref = refs[:4]
            k = 4
        else:
            x_ref, id_ref, q_ref, lo_ref, hi_ref = refs[:5]
            k = 5
        m_ref, s_ref, r_ref, m_acc, s_acc, r_acc = refs[k:k + 6]
        i = pl.program_id(0)
        if node_mode:
            ids = i * t + lax.broadcasted_iota(I32, (t, 1), 0)
        else:
            ids = id_ref[...]
        oh = ((ids >= lo_ref[...]) & (ids < hi_ref[...])).astype(F32)
        xv = x_ref[...]
        qe = jnp.dot(oh, q_ref[...])
        e = jnp.sum(xv * qe, axis=1, keepdims=True)
        masked = oh * e - (1.0 - oh) * 1e30
        mt = jnp.max(masked, axis=0, keepdims=True)

        @pl.when(i == 0)
        def _():
            m_acc[...] = jnp.full_like(m_acc, -1e30)
            s_acc[...] = jnp.zeros_like(s_acc)
            r_acc[...] = jnp.zeros_like(r_acc)

        m_old = m_acc[...]
        m_new = jnp.maximum(m_old, mt)
        scale = jnp.exp(m_old - m_new)
        m_e = jnp.sum(oh * m_new, axis=1, keepdims=True)
        p = jnp.exp(e - m_e)
        w = oh * p
        s_acc[...] = s_acc[...] * scale + jnp.sum(w, axis=0, keepdims=True)
        r_acc[...] = r_acc[...] * scale + lax.dot_general(
            xv, w, (((0,), (0,)), ((), ())))
        m_acc[...] = m_new

        @pl.when(i == G - 1)
        def _():
            m_ref[...] = m_acc[...]
            s_ref[...] = s_acc[...]
            r_ref[...] = r_acc[...]

    ins = [xx] + ([] if node_mode else [ids_col]) + [q, lo, hi]
    in_specs = [_row_spec(t, 16)] + ([] if node_mode else [_row_spec(t, 1)]) \
        + [_full(a) for a in (q, lo, hi)]
    return pl.pallas_call(
        body, grid=(G,),
        in_specs=in_specs,
        out_specs=[_full(jnp.zeros((1, 64))), _full(jnp.zeros((1, 64))),
                   _full(jnp.zeros((16, 64)))],
        out_shape=[jax.ShapeDtypeStruct((1, 64), F32),
                   jax.ShapeDtypeStruct((1, 64), F32),
                   jax.ShapeDtypeStruct((16, 64), F32)],
        scratch_shapes=[pltpu.VMEM((1, 64), F32), pltpu.VMEM((1, 64), F32),
                        pltpu.VMEM((16, 64), F32)],
    )(*ins)


def _sc_mesh():
    return plsc.VectorSubcoreMesh(core_axis_name="c", subcore_axis_name="s")


def _sc_gather2(tab, srcp, dstp):
    re_rows = srcp.shape[0]
    epad = re_rows * _LW
    ch = 8
    rpw = re_rows // _NW
    nb = rpw // ch
    eb = ch * _LW

    def body(tab_hbm, si_hbm, di_hbm, vs_hbm, vd_hbm,
             idx0, idx1, dat0, dat1, gsem, wsem0, wsem1, isem):
        c = lax.axis_index("c")
        s = lax.axis_index("s")
        wid = s * _NC + c
        base = wid * rpw
        idxs = (idx0, idx1)
        dats = (dat0, dat1)
        wsems = (wsem0, wsem1)

        def direction(idx_hbm, out_hbm):
            pltpu.sync_copy(idx_hbm.at[pl.ds(base, ch)], idx0)

            def pair(g, carry):
                for b in range(2):
                    j = g * 2 + b
                    idx_v = idxs[b]
                    dat_v = dats[b]
                    rb = base + j * ch

                    @pl.when(j >= 2)
                    def _():
                        pltpu.make_async_copy(
                            out_hbm.at[pl.ds(0, eb)], dat_v, wsems[b]).wait()

                    @pl.when(j >= 1)
                    def _():
                        pltpu.make_async_copy(
                            idx_hbm.at[pl.ds(base, ch)], idx_v, isem).wait()

                    hs = [pltpu.async_copy(tab_hbm.at[idx_v.at[k]],
                                           dat_v.at[pl.ds(k * _LW, _LW)], gsem)
                          for k in range(ch)]
                    nxt = jnp.minimum(j + 1, nb - 1)
                    pltpu.async_copy(idx_hbm.at[pl.ds(base + nxt * ch, ch)],
                                     idxs[1 - b], isem)
                    for h in hs:
                        h.wait()
                    pltpu.async_copy(dat_v, out_hbm.at[pl.ds(rb * _LW, eb)],
                                     wsems[b])
                return carry

            lax.fori_loop(0, nb // 2, pair, 0)
            pltpu.make_async_copy(idx_hbm.at[pl.ds(base, ch)],
                                  idxs[nb % 2], isem).wait()
            pltpu.make_async_copy(out_hbm.at[pl.ds(0, eb)], dat0, wsem0).wait()
            pltpu.make_async_copy(out_hbm.at[pl.ds(0, eb)], dat1, wsem1).wait()

        direction(si_hbm, vs_hbm)
        direction(di_hbm, vd_hbm)

    f = pl.kernel(
        body,
        out_type=[jax.ShapeDtypeStruct((epad, 32), F32),
                  jax.ShapeDtypeStruct((epad, 32), F32)],
        mesh=_sc_mesh(),
        compiler_params=pltpu.CompilerParams(use_tc_tiling_on_sc=False),
        scratch_types=[pltpu.VMEM((ch, _LW), I32),
                       pltpu.VMEM((ch, _LW), I32),
                       pltpu.VMEM((eb, 32), F32),
                       pltpu.VMEM((eb, 32), F32),
                       pltpu.SemaphoreType.DMA,
                       pltpu.SemaphoreType.DMA,
                       pltpu.SemaphoreType.DMA,
                       pltpu.SemaphoreType.DMA],
    )
    return f(tab, srcp, dstp)


def _sc_scatter(ep, dstp, npad):
    ch = 4
    re_rows = dstp.shape[0]
    rpt = re_rows // _NS
    nb = rpt // ch
    eb = ch * _LW
    rt = npad // _NS
    zch = rt // 16

    def body(ep_hbm, di_hbm, out_hbm, acc_sh,
             idx0, idx1, dat0, dat1, dsem, asem0, asem1):
        c = lax.axis_index("c")
        s = lax.axis_index("s")
        idxs = (idx0, idx1)
        dats = (dat0, dat1)
        asems = (asem0, asem1)

        def zb(i, carry):
            dat0[i, :] = jnp.zeros((16,), F32)
            return carry

        lax.fori_loop(0, zch, zb, 0)

        def zc(k, carry):
            pltpu.sync_copy(dat0.at[pl.ds(0, zch)],
                            acc_sh.at[pl.ds(s * rt + k * zch, zch)])
            return carry

        lax.fori_loop(0, 16, zc, 0)
        plsc.subcore_barrier()

        base = s * rpt
        pltpu.sync_copy(di_hbm.at[pl.ds(base, ch)], idx0)
        pltpu.sync_copy(ep_hbm.at[pl.ds(base * _LW, eb), pl.ds(c * 16, 16)],
                        dat0)

        def pair(g, carry):
            for b in range(2):
                j = g * 2 + b
                idx_v = idxs[b]
                dat_v = dats[b]

                @pl.when(j >= 1)
                def _():
                    pltpu.make_async_copy(di_hbm.at[pl.ds(base, ch)],
                                          idx_v, dsem).wait()
                    pltpu.make_async_copy(ep_hbm.at[pl.ds(0, eb),
                                                    pl.ds(c * 16, 16)],
                                          dat_v, dsem).wait()

                for k in range(ch):
                    pltpu.async_copy(dat_v.at[pl.ds(k * _LW, _LW)],
                                     acc_sh.at[idx_v.at[k]], asems[b],
                                     add=True)

                @pl.when(j >= 1)
                def _():
                    pltpu.make_async_copy(ep_hbm.at[pl.ds(0, eb),
                                                    pl.ds(c * 16, 16)],
                                          dats[1 - b], asems[1 - b]).wait()

                nxt = jnp.minimum(j + 1, nb - 1)
                rb = base + nxt * ch
                pltpu.async_copy(di_hbm.at[pl.ds(rb, ch)], idxs[1 - b], dsem)
                pltpu.async_copy(ep_hbm.at[pl.ds(rb * _LW, eb),
                                           pl.ds(c * 16, 16)],
                                 dats[1 - b], dsem)
            return carry

        lax.fori_loop(0, nb // 2, pair, 0)
        pltpu.make_async_copy(di_hbm.at[pl.ds(base, ch)],
                              idxs[nb % 2], dsem).wait()
        pltpu.make_async_copy(ep_hbm.at[pl.ds(0, eb), pl.ds(c * 16, 16)],
                              dats[nb % 2], dsem).wait()
        pltpu.make_async_copy(ep_hbm.at[pl.ds(0, eb), pl.ds(c * 16, 16)],
                              dats[(nb - 1) % 2], asems[(nb - 1) % 2]).wait()
        plsc.subcore_barrier()

        def wb(k, carry):
            r0 = s * rt + k * zch
            pltpu.sync_copy(acc_sh.at[pl.ds(r0, zch)], dat0.at[pl.ds(0, zch)])
            pltpu.sync_copy(dat0.at[pl.ds(0, zch)],
                            out_hbm.at[pl.ds(r0, zch), pl.ds(c * 16, 16)])
            return carry

        lax.fori_loop(0, 16, wb, 0)

    f = pl.kernel(
        body,
        out_type=jax.ShapeDtypeStruct((npad, 32), F32),
        mesh=_sc_mesh(),
        compiler_params=pltpu.CompilerParams(use_tc_tiling_on_sc=False),
        scratch_types=[pltpu.VMEM_SHARED((npad, 16), F32),
                       pltpu.VMEM((ch, _LW), I32),
                       pltpu.VMEM((ch, _LW), I32),
                       pltpu.VMEM((eb, 16), F32),
                       pltpu.VMEM((eb, 16), F32),
                       pltpu.SemaphoreType.DMA,
                       pltpu.SemaphoreType.DMA,
                       pltpu.SemaphoreType.DMA],
    )
    return f(ep, dstp)


def _sc_counts(dstp, npad):
    re_rows = dstp.shape[0]
    rpsc = re_rows // _NC
    rpt = rpsc // _NS
    nb = rpt // _CH
    rt = npad // _NS

    def body(di_hbm, out_hbm, acc_sh, idx_v, one_v, sem):
        c = lax.axis_index("c")
        s = lax.axis_index("s")

        def zb(i, carry):
            one_v[i, :] = jnp.zeros((16,), F32)
            return carry

        lax.fori_loop(0, _LW, zb, 0)

        def zc(k, carry):
            pltpu.sync_copy(one_v, acc_sh.at[pl.ds(s * rt + k * _LW, _LW)])
            return carry

        lax.fori_loop(0, rt // _LW, zc, 0)
        plsc.subcore_barrier()

        def ob(i, carry):
            one_v[i, :] = jnp.ones((16,), F32)
            return carry

        lax.fori_loop(0, _LW, ob, 0)

        def loop(j2, carry):
            rb = c * rpsc + s * rpt + j2 * _CH
            pltpu.sync_copy(di_hbm.at[pl.ds(rb, _CH)], idx_v)
            hs = [pltpu.async_copy(one_v, acc_sh.at[idx_v.at[j]], sem, add=True)
                  for j in range(_CH)]
            for h in hs:
                h.wait()
            return carry

        lax.fori_loop(0, nb, loop, 0)
        plsc.subcore_barrier()

        def wb(k, carry):
            r0 = s * rt + k * _LW
            pltpu.sync_copy(acc_sh.at[pl.ds(r0, _LW)], one_v)
            pltpu.sync_copy(one_v, out_hbm.at[c, pl.ds(r0, _LW)])
            return carry

        lax.fori_loop(0, rt // _LW, wb, 0)

    f = pl.kernel(
        body,
        out_type=jax.ShapeDtypeStruct((_NC, npad, 16), F32),
        mesh=_sc_mesh(),
        compiler_params=pltpu.CompilerParams(use_tc_tiling_on_sc=False),
        scratch_types=[pltpu.VMEM_SHARED((npad, 16), F32),
                       pltpu.VMEM((_CH, _LW), I32),
                       pltpu.VMEM((_LW, 16), F32),
                       pltpu.SemaphoreType.DMA],
    )
    return f(dstp)


def _lin_j(x, wb):
    w, b = wb
    return jnp.dot(x, w.T) + b


def _ff_j(x, pp):
    return jax.nn.softplus(_lin_j(jax.nn.softplus(_lin_j(x, pp[0])), pp[1]))


def _set2set(xx, ids_col, lp, lo, hi, t):
    w_ih, w_hh, b_ih, b_hh = lp
    q_star = jnp.zeros((64, 32), F32)
    h = jnp.zeros((64, 16), F32)
    cst = jnp.zeros((64, 16), F32)
    for _ in range(3):
        gates = jnp.dot(q_star, w_ih.T) + b_ih + jnp.dot(h, w_hh.T) + b_hh
        ig, fg, gg, og = jnp.split(gates, 4, axis=1)
        ig = jax.nn.sigmoid(ig)
        fg = jax.nn.sigmoid(fg)
        gg = jnp.tanh(gg)
        og = jax.nn.sigmoid(og)
        cst = fg * cst + ig * gg
        h = og * jnp.tanh(cst)
        q = h
        _, sexp, r = _s2s_pass(xx, ids_col, q, lo, hi, t)
        rg = (r / (sexp + 1e-16)).T
        q_star = jnp.concatenate([q, rg], axis=1)
    return q_star


def _t2(wb):
    w, b = wb
    return w.T, b.reshape(1, -1)


def kernel(x, edge_index, edge_attr, batch, params):
    p = params
    N = x.shape[0]
    E = edge_index.shape[1]
    unit = _NW * _CH * _LW
    epad = ((E + unit - 1) // unit) * unit
    npad = (N // 512 + 1) * 512
    trash = npad - 1
    re_rows = epad // _LW

    src = edge_index[0].astype(I32)
    dst = edge_index[1].astype(I32)
    padv = jnp.full((epad - E,), trash, I32)
    srcp = jnp.concatenate([src, padv]).reshape(re_rows, _LW)
    dstp = jnp.concatenate([dst, padv]).reshape(re_rows, _LW)
    src_col = src.reshape(E, 1)
    dst_col = dst.reshape(E, 1)

    starts = jnp.searchsorted(batch, jnp.arange(65, dtype=I32)).astype(I32)
    lo = starts[:64].reshape(1, 64)
    hi = starts[1:].reshape(1, 64)
    bcnt = jnp.maximum((starts[1:] - starts[:64]).astype(F32), 1.0).reshape(64, 1)

    e0 = _edge_init(edge_attr, *_t2(p['ffe0'][0]), *_t2(p['ffe0'][1]))
    v0 = _node_init(x, *_t2(p['embedding']), *_t2(p['ffv0'][0]), *_t2(p['ffv0'][1]))
    u0 = _ff_j(jnp.zeros((64, 2), F32), p['ffu0'])

    cnts = _sc_counts(dstp, npad)
    c0 = cnts[0]
    c1 = cnts[1]

    vres, eres, ures = v0, e0, u0
    for li, mk in enumerate(('meg1', 'meg2', 'meg3')):
        mp = p[mk]
        if li == 0:
            vff, uin, ew = vres, ures, None
        else:
            fv, fe, fu = p['ffv%d' % li], p['ffe%d' % li], p['ffu%d' % li]
            vff = _node_ff(vres, *_t2(fv[0]), *_t2(fv[1]))
            ew = (*_t2(fe[0]), *_t2(fe[1]))
            uin = _ff_j(ures, fu)
        tab = jnp.pad(vff, ((0, npad - N), (0, 0)))
        vs, vd = _sc_gather2(tab, srcp, dstp)
        cw_e = (*_t2(mp['e1']), *_t2(mp['e2']), *_t2(mp['e3']))
        ep, enext = _edge_conv(eres, vs, vd, src_col, lo, hi, uin, ew, cw_e, epad)
        sums = _sc_scatter(ep, dstp, npad)
        cw_v = (*_t2(mp['v1']), *_t2(mp['v2']), *_t2(mp['v3']))
        vnext, ue_s, uv_s = _node_conv(vres, vff, sums, c0, c1, uin, lo, hi, cw_v)
        ue = ue_s / bcnt
        uv = uv_s / bcnt
        uc = jnp.concatenate([ue, uv, uin], axis=1)
        upd = jax.nn.softplus(_lin_j(uc, mp['u1']))
        upd = jax.nn.softplus(_lin_j(upd, mp['u2']))
        upd = jax.nn.softplus(_lin_j(upd, mp['u3']))
        vres, eres, ures = vnext, enext, upd + ures

    xv = _linear16(vres, *_t2(p['node_linear']), _TN)
    xe = _linear16(eres, *_t2(p['edge_linear']), _TE)
    node_vec = _set2set(xv, None, p['lstm_node'], lo, hi, _TN)
    edge_vec = _set2set(xe, dst_col, p['lstm_edge'], lo, hi, _TE)
    fin = jnp.concatenate([node_vec, edge_vec, ures], axis=1)
    out = jax.nn.softplus(_lin_j(fin, p['fc1']))
    out = jax.nn.softplus(_lin_j(out, p['fc2']))
    return _lin_j(out, p['fc3'])


# speedup vs baseline: 6.5681x; 1.4732x over previous
import functools

import jax
import jax.numpy as jnp
from jax import lax
from jax.experimental import pallas as pl
from jax.experimental.pallas import tpu as pltpu
from jax.experimental.pallas import tpu_sc as plsc

F32 = jnp.float32
I32 = jnp.int32

_NC = 2
_NS = 16
_NW = _NC * _NS
_LW = 128
_CH = 16

_TE = 4000
_TN = 2000


def _sp(x):
    return jnp.maximum(x, 0.0) + jnp.log(1.0 + jnp.exp(-jnp.abs(x)))


def _full(a):
    return pl.BlockSpec(a.shape, lambda i: (0,) * a.ndim)


def _row_spec(t, w):
    return pl.BlockSpec((t, w), lambda i: (i, 0))


def _edge_init(edge_attr, w1t, b1, w2t, b2):
    E = edge_attr.shape[0]
    G = E // _TE

    def body(a_ref, w1_ref, b1_ref, w2_ref, b2_ref, o_ref):
        a = a_ref[...]
        d = jnp.sqrt(jnp.sum(a * a, axis=1, keepdims=True))
        cen = lax.broadcasted_iota(I32, (1, 100), 1).astype(F32) * (5.0 / 99.0)
        r = jnp.exp(-4.0 * (d - cen) ** 2)
        h = _sp(jnp.dot(r, w1_ref[...]) + b1_ref[...])
        o_ref[...] = _sp(jnp.dot(h, w2_ref[...]) + b2_ref[...])

    return pl.pallas_call(
        body, grid=(G,),
        in_specs=[_row_spec(_TE, 3), _full(w1t), _full(b1), _full(w2t), _full(b2)],
        out_specs=_row_spec(_TE, 32),
        out_shape=jax.ShapeDtypeStruct((E, 32), F32),
    )(edge_attr, w1t, b1, w2t, b2)


def _node_init(x, wet, be, w1t, b1, w2t, b2):
    N = x.shape[0]
    G = N // _TN

    def body(x_ref, we_ref, be_ref, w1_ref, b1_ref, w2_ref, b2_ref, o_ref):
        v = jnp.dot(x_ref[...], we_ref[...]) + be_ref[...]
        h = _sp(jnp.dot(v, w1_ref[...]) + b1_ref[...])
        o_ref[...] = _sp(jnp.dot(h, w2_ref[...]) + b2_ref[...])

    return pl.pallas_call(
        body, grid=(G,),
        in_specs=[_row_spec(_TN, x.shape[1])] + [_full(a) for a in (wet, be, w1t, b1, w2t, b2)],
        out_specs=_row_spec(_TN, 32),
        out_shape=jax.ShapeDtypeStruct((N, 32), F32),
    )(x, wet, be, w1t, b1, w2t, b2)


def _node_ff(v, w1t, b1, w2t, b2):
    N = v.shape[0]
    G = N // _TN

    def body(v_ref, w1_ref, b1_ref, w2_ref, b2_ref, o_ref):
        h = _sp(jnp.dot(v_ref[...], w1_ref[...]) + b1_ref[...])
        o_ref[...] = _sp(jnp.dot(h, w2_ref[...]) + b2_ref[...])

    return pl.pallas_call(
        body, grid=(G,),
        in_specs=[_row_spec(_TN, 32)] + [_full(a) for a in (w1t, b1, w2t, b2)],
        out_specs=_row_spec(_TN, 32),
        out_shape=jax.ShapeDtypeStruct((N, 32), F32),
    )(v, w1t, b1, w2t, b2)


def _edge_conv(eres, vs, vd, src_col, lo, hi, u, ffw, cw, epad):
    E = eres.shape[0]
    tp = _TE // 4
    G = E // _TE
    has_ff = ffw is not None

    erp = eres.reshape(E // 4, 128)
    vsp = vs.reshape(epad // 4, 128)
    vdp = vd.reshape(epad // 4, 128)
    scp = src_col.reshape(E // 4, 4)

    k4 = functools.partial(jnp.kron, jnp.eye(4, dtype=F32))
    t4 = functools.partial(jnp.tile, reps=(1, 4))
    ones64 = k4(jnp.ones((1, 64), F32))
    lo4 = t4(lo.astype(F32))
    hi4 = t4(hi.astype(F32))

    if has_ff:
        fw1, fb1, fw2, fb2 = ffw
        ffw_p = (k4(fw1), t4(fb1), k4(fw2), t4(fb2))
    w1, b1, w2, b2, w3, b3 = cw
    cw_p = (k4(w1[:32]), k4(w1[32:64]), k4(w1[64:96]), k4(w1[96:]), t4(b1),
            k4(w2), t4(b2), k4(w3), t4(b3))

    def body(*refs):
        e_ref, vs_ref, vd_ref, s_ref, o64_ref, lo_ref, hi_ref, u_ref = refs[:8]
        k = 8
        if has_ff:
            fw1p, fb1p, fw2p, fb2p = (r[...] for r in refs[k:k + 4])
            k += 4
        w1a, w1b, w1c, w1d, bb1, w2p, bb2, w3p, bb3 = \
            (r[...] for r in refs[k:k + 9])
        ep_ref, en_ref = refs[k + 9:k + 11]
        er = e_ref[...]
        if has_ff:
            eff = _sp(jnp.dot(_sp(jnp.dot(er, fw1p) + fb1p), fw2p) + fb2p)
        else:
            eff = er
        s_rep = jnp.dot(s_ref[...].astype(F32), o64_ref[...])
        oh = ((s_rep >= lo_ref[...]) & (s_rep < hi_ref[...])).astype(F32)
        ub = jnp.dot(oh, u_ref[...])
        h = _sp(jnp.dot(vs_ref[...], w1a) + jnp.dot(vd_ref[...], w1b)
                + jnp.dot(ub, w1c) + jnp.dot(eff, w1d) + bb1)
        h = _sp(jnp.dot(h, w2p) + bb2)
        ep = _sp(jnp.dot(h, w3p) + bb3)
        ep_ref[...] = ep
        en_ref[...] = ep + er

    ins = [erp, vsp, vdp, scp, ones64, lo4, hi4, k4(u)] \
        + (list(ffw_p) if has_ff else []) + list(cw_p)
    in_specs = ([_row_spec(tp, 128)] * 3 + [_row_spec(tp, 4)]
                + [_full(a) for a in ins[4:]])
    ep_p, en_p = pl.pallas_call(
        body, grid=(G,),
        in_specs=in_specs,
        out_specs=[_row_spec(tp, 128)] * 2,
        out_shape=[jax.ShapeDtypeStruct((epad // 4, 128), F32),
                   jax.ShapeDtypeStruct((E // 4, 128), F32)],
    )(*ins)
    return ep_p.reshape(epad, 32), en_p.reshape(E, 32)


def _node_conv(vres, vff, sums, c0, c1, u, lo, hi, cw):
    N = vres.shape[0]
    G = N // _TN

    def body(vr_ref, vf_ref, sm_ref, c0_ref, c1_ref, u_ref, lo_ref, hi_ref,
             w1_ref, b1_ref, w2_ref, b2_ref, w3_ref, b3_ref,
             vn_ref, ue_ref, uv_ref, ue_acc, uv_acc):
        i = pl.program_id(0)
        nid = i * _TN + lax.broadcasted_iota(I32, (_TN, 1), 0)
        oh = ((nid >= lo_ref[...]) & (nid < hi_ref[...])).astype(F32)
        cnt = jnp.maximum(c0_ref[...][:, :1] + c1_ref[...][:, :1], 1.0)
        etv = sm_ref[...] / cnt
        ub = jnp.dot(oh, u_ref[...])
        cc = jnp.concatenate([vf_ref[...], etv, ub], axis=1)
        h = _sp(jnp.dot(cc, w1_ref[...]) + b1_ref[...])
        h = _sp(jnp.dot(h, w2_ref[...]) + b2_ref[...])
        vp = _sp(jnp.dot(h, w3_ref[...]) + b3_ref[...])
        vn_ref[...] = vp + vr_ref[...]

        @pl.when(i == 0)
        def _():
            ue_acc[...] = jnp.zeros_like(ue_acc)
            uv_acc[...] = jnp.zeros_like(uv_acc)

        dn = (((0,), (0,)), ((), ()))
        ue_acc[...] += lax.dot_general(oh, etv, dn)
        uv_acc[...] += lax.dot_general(oh, vp, dn)

        @pl.when(i == G - 1)
        def _():
            ue_ref[...] = ue_acc[...]
            uv_ref[...] = uv_acc[...]

    ins = [vres, vff, sums, c0, c1, u, lo, hi] + list(cw)
    in_specs = ([_row_spec(_TN, 32)] * 3 + [_row_spec(_TN, 16)] * 2
                + [_full(a) for a in ins[5:]])
    return pl.pallas_call(
        body, grid=(G,),
        in_specs=in_specs,
        out_specs=[_row_spec(_TN, 32), _full(jnp.zeros((64, 32))), _full(jnp.zeros((64, 32)))],
        out_shape=[jax.ShapeDtypeStruct((N, 32), F32),
                   jax.ShapeDtypeStruct((64, 32), F32),
                   jax.ShapeDtypeStruct((64, 32), F32)],
        scratch_shapes=[pltpu.VMEM((64, 32), F32), pltpu.VMEM((64, 32), F32)],
    )(*ins)


def _linear16(x, wt, b, t):
    M = x.shape[0]
    G = M // t

    def body(x_ref, w_ref, b_ref, o_ref):
        o_ref[...] = jnp.dot(x_ref[...], w_ref[...]) + b_ref[...]

    return pl.pallas_call(
        body, grid=(G,),
        in_specs=[_row_spec(t, 32), _full(wt), _full(b)],
        out_specs=_row_spec(t, 16),
        out_shape=jax.ShapeDtypeStruct((M, 16), F32),
    )(x, wt, b)


def _s2s_pass(xx, ids_col, q, lo, hi, t):
    M = xx.shape[0]
    G = M // t
    node_mode = ids_col is None

    def body(*refs):
        if node_mode:
            x_ref, q_ref, lo_ref, hi_ref = refs[:4]
            k = 4
        else:
            x_ref, id_ref, q_ref, lo_ref, hi_ref = refs[:5]
            k = 5
        m_ref, s_ref, r_ref, m_acc, s_acc, r_acc = refs[k:k + 6]
        i = pl.program_id(0)
        if node_mode:
            ids = i * t + lax.broadcasted_iota(I32, (t, 1), 0)
        else:
            ids = id_ref[...]
        oh = ((ids >= lo_ref[...]) & (ids < hi_ref[...])).astype(F32)
        xv = x_ref[...]
        qe = jnp.dot(oh, q_ref[...])
        e = jnp.sum(xv * qe, axis=1, keepdims=True)
        masked = oh * e - (1.0 - oh) * 1e30
        mt = jnp.max(masked, axis=0, keepdims=True)

        @pl.when(i == 0)
        def _():
            m_acc[...] = jnp.full_like(m_acc, -1e30)
            s_acc[...] = jnp.zeros_like(s_acc)
            r_acc[...] = jnp.zeros_like(r_acc)

        m_old = m_acc[...]
        m_new = jnp.maximum(m_old, mt)
        scale = jnp.exp(m_old - m_new)
        m_e = jnp.sum(oh * m_new, axis=1, keepdims=True)
        p = jnp.exp(e - m_e)
        w = oh * p
        s_acc[...] = s_acc[...] * scale + jnp.sum(w, axis=0, keepdims=True)
        r_acc[...] = r_acc[...] * scale + lax.dot_general(
            xv, w, (((0,), (0,)), ((), ())))
        m_acc[...] = m_new

        @pl.when(i == G - 1)
        def _():
            m_ref[...] = m_acc[...]
            s_ref[...] = s_acc[...]
            r_ref[...] = r_acc[...]

    ins = [xx] + ([] if node_mode else [ids_col]) + [q, lo, hi]
    in_specs = [_row_spec(t, 16)] + ([] if node_mode else [_row_spec(t, 1)]) \
        + [_full(a) for a in (q, lo, hi)]
    return pl.pallas_call(
        body, grid=(G,),
        in_specs=in_specs,
        out_specs=[_full(jnp.zeros((1, 64))), _full(jnp.zeros((1, 64))),
                   _full(jnp.zeros((16, 64)))],
        out_shape=[jax.ShapeDtypeStruct((1, 64), F32),
                   jax.ShapeDtypeStruct((1, 64), F32),
                   jax.ShapeDtypeStruct((16, 64), F32)],
        scratch_shapes=[pltpu.VMEM((1, 64), F32), pltpu.VMEM((1, 64), F32),
                        pltpu.VMEM((16, 64), F32)],
    )(*ins)


def _sc_mesh():
    return plsc.VectorSubcoreMesh(core_axis_name="c", subcore_axis_name="s")


def _sc_gather2(tab, srcp, dstp):
    re_rows = srcp.shape[0]
    epad = re_rows * _LW
    ch = 8
    rpw = re_rows // _NW
    nb = rpw // ch
    eb = ch * _LW

    def body(tab_hbm, si_hbm, di_hbm, vs_hbm, vd_hbm,
             idx0, idx1, dat0, dat1, gsem, wsem0, wsem1, isem):
        c = lax.axis_index("c")
        s = lax.axis_index("s")
        wid = s * _NC + c
        base = wid * rpw
        idxs = (idx0, idx1)
        dats = (dat0, dat1)
        wsems = (wsem0, wsem1)

        def direction(idx_hbm, out_hbm):
            pltpu.sync_copy(idx_hbm.at[pl.ds(base, ch)], idx0)

            def pair(g, carry):
                for b in range(2):
                    j = g * 2 + b
                    idx_v = idxs[b]
                    dat_v = dats[b]
                    rb = base + j * ch

                    @pl.when(j >= 2)
                    def _():
                        pltpu.make_async_copy(
                            out_hbm.at[pl.ds(0, eb)], dat_v, wsems[b]).wait()

                    @pl.when(j >= 1)
                    def _():
                        pltpu.make_async_copy(
                            idx_hbm.at[pl.ds(base, ch)], idx_v, isem).wait()

                    hs = [pltpu.async_copy(tab_hbm.at[idx_v.at[k]],
                                           dat_v.at[pl.ds(k * _LW, _LW)], gsem)
                          for k in range(ch)]
                    nxt = jnp.minimum(j + 1, nb - 1)
                    pltpu.async_copy(idx_hbm.at[pl.ds(base + nxt * ch, ch)],
                                     idxs[1 - b], isem)
                    for h in hs:
                        h.wait()
                    pltpu.async_copy(dat_v, out_hbm.at[pl.ds(rb * _LW, eb)],
                                     wsems[b])
                return carry

            lax.fori_loop(0, nb // 2, pair, 0)
            pltpu.make_async_copy(idx_hbm.at[pl.ds(base, ch)],
                                  idxs[nb % 2], isem).wait()
            pltpu.make_async_copy(out_hbm.at[pl.ds(0, eb)], dat0, wsem0).wait()
            pltpu.make_async_copy(out_hbm.at[pl.ds(0, eb)], dat1, wsem1).wait()

        direction(si_hbm, vs_hbm)
        direction(di_hbm, vd_hbm)

    f = pl.kernel(
        body,
        out_type=[jax.ShapeDtypeStruct((epad, 32), F32),
                  jax.ShapeDtypeStruct((epad, 32), F32)],
        mesh=_sc_mesh(),
        compiler_params=pltpu.CompilerParams(use_tc_tiling_on_sc=False),
        scratch_types=[pltpu.VMEM((ch, _LW), I32),
                       pltpu.VMEM((ch, _LW), I32),
                       pltpu.VMEM((eb, 32), F32),
                       pltpu.VMEM((eb, 32), F32),
                       pltpu.SemaphoreType.DMA,
                       pltpu.SemaphoreType.DMA,
                       pltpu.SemaphoreType.DMA,
                       pltpu.SemaphoreType.DMA],
    )
    return f(tab, srcp, dstp)


def _sc_scatter(ep, dstp, npad):
    ch = 4
    re_rows = dstp.shape[0]
    rpt = re_rows // _NS
    nb = rpt // ch
    eb = ch * _LW
    rt = npad // _NS
    zch = rt // 16

    def body(ep_hbm, di_hbm, out_hbm, acc_sh,
             idx0, idx1, dat0, dat1, dsem, asem0, asem1):
        c = lax.axis_index("c")
        s = lax.axis_index("s")
        idxs = (idx0, idx1)
        dats = (dat0, dat1)
        asems = (asem0, asem1)

        def zb(i, carry):
            dat0[i, :] = jnp.zeros((16,), F32)
            return carry

        lax.fori_loop(0, zch, zb, 0)

        def zc(k, carry):
            pltpu.sync_copy(dat0.at[pl.ds(0, zch)],
                            acc_sh.at[pl.ds(s * rt + k * zch, zch)])
            return carry

        lax.fori_loop(0, 16, zc, 0)
        plsc.subcore_barrier()

        base = s * rpt
        pltpu.sync_copy(di_hbm.at[pl.ds(base, ch)], idx0)
        pltpu.sync_copy(ep_hbm.at[pl.ds(base * _LW, eb), pl.ds(c * 16, 16)],
                        dat0)

        def pair(g, carry):
            for b in range(2):
                j = g * 2 + b
                idx_v = idxs[b]
                dat_v = dats[b]

                @pl.when(j >= 1)
                def _():
                    pltpu.make_async_copy(di_hbm.at[pl.ds(base, ch)],
                                          idx_v, dsem).wait()
                    pltpu.make_async_copy(ep_hbm.at[pl.ds(0, eb),
                                                    pl.ds(c * 16, 16)],
                                          dat_v, dsem).wait()

                for k in range(ch):
                    pltpu.async_copy(dat_v.at[pl.ds(k * _LW, _LW)],
                                     acc_sh.at[idx_v.at[k]], asems[b],
                                     add=True)

                @pl.when(j >= 1)
                def _():
                    pltpu.make_async_copy(ep_hbm.at[pl.ds(0, eb),
                                                    pl.ds(c * 16, 16)],
                                          dats[1 - b], asems[1 - b]).wait()

                nxt = jnp.minimum(j + 1, nb - 1)
                rb = base + nxt * ch
                pltpu.async_copy(di_hbm.at[pl.ds(rb, ch)], idxs[1 - b], dsem)
                pltpu.async_copy(ep_hbm.at[pl.ds(rb * _LW, eb),
                                           pl.ds(c * 16, 16)],
                                 dats[1 - b], dsem)
            return carry

        lax.fori_loop(0, nb // 2, pair, 0)
        pltpu.make_async_copy(di_hbm.at[pl.ds(base, ch)],
                              idxs[nb % 2], dsem).wait()
        pltpu.make_async_copy(ep_hbm.at[pl.ds(0, eb), pl.ds(c * 16, 16)],
                              dats[nb % 2], dsem).wait()
        pltpu.make_async_copy(ep_hbm.at[pl.ds(0, eb), pl.ds(c * 16, 16)],
                              dats[(nb - 1) % 2], asems[(nb - 1) % 2]).wait()
        plsc.subcore_barrier()

        def wb(k, carry):
            r0 = s * rt + k * zch
            pltpu.sync_copy(acc_sh.at[pl.ds(r0, zch)], dat0.at[pl.ds(0, zch)])
            pltpu.sync_copy(dat0.at[pl.ds(0, zch)],
                            out_hbm.at[pl.ds(r0, zch), pl.ds(c * 16, 16)])
            return carry

        lax.fori_loop(0, 16, wb, 0)

    f = pl.kernel(
        body,
        out_type=jax.ShapeDtypeStruct((npad, 32), F32),
        mesh=_sc_mesh(),
        compiler_params=pltpu.CompilerParams(use_tc_tiling_on_sc=False),
        scratch_types=[pltpu.VMEM_SHARED((npad, 16), F32),
                       pltpu.VMEM((ch, _LW), I32),
                       pltpu.VMEM((ch, _LW), I32),
                       pltpu.VMEM((eb, 16), F32),
                       pltpu.VMEM((eb, 16), F32),
                       pltpu.SemaphoreType.DMA,
                       pltpu.SemaphoreType.DMA,
                       pltpu.SemaphoreType.DMA],
    )
    return f(ep, dstp)


def _sc_counts(dstp, npad):
    re_rows = dstp.shape[0]
    rpsc = re_rows // _NC
    rpt = rpsc // _NS
    nb = rpt // _CH
    rt = npad // _NS

    def body(di_hbm, out_hbm, acc_sh, idx_v, one_v, sem):
        c = lax.axis_index("c")
        s = lax.axis_index("s")

        def zb(i, carry):
            one_v[i, :] = jnp.zeros((16,), F32)
            return carry

        lax.fori_loop(0, _LW, zb, 0)

        def zc(k, carry):
            pltpu.sync_copy(one_v, acc_sh.at[pl.ds(s * rt + k * _LW, _LW)])
            return carry

        lax.fori_loop(0, rt // _LW, zc, 0)
        plsc.subcore_barrier()

        def ob(i, carry):
            one_v[i, :] = jnp.ones((16,), F32)
            return carry

        lax.fori_loop(0, _LW, ob, 0)

        def loop(j2, carry):
            rb = c * rpsc + s * rpt + j2 * _CH
            pltpu.sync_copy(di_hbm.at[pl.ds(rb, _CH)], idx_v)
            hs = [pltpu.async_copy(one_v, acc_sh.at[idx_v.at[j]], sem, add=True)
                  for j in range(_CH)]
            for h in hs:
                h.wait()
            return carry

        lax.fori_loop(0, nb, loop, 0)
        plsc.subcore_barrier()

        def wb(k, carry):
            r0 = s * rt + k * _LW
            pltpu.sync_copy(acc_sh.at[pl.ds(r0, _LW)], one_v)
            pltpu.sync_copy(one_v, out_hbm.at[c, pl.ds(r0, _LW)])
            return carry

        lax.fori_loop(0, rt // _LW, wb, 0)

    f = pl.kernel(
        body,
        out_type=jax.ShapeDtypeStruct((_NC, npad, 16), F32),
        mesh=_sc_mesh(),
        compiler_params=pltpu.CompilerParams(use_tc_tiling_on_sc=False),
        scratch_types=[pltpu.VMEM_SHARED((npad, 16), F32),
                       pltpu.VMEM((_CH, _LW), I32),
                       pltpu.VMEM((_LW, 16), F32),
                       pltpu.SemaphoreType.DMA],
    )
    return f(dstp)


def _lin_j(x, wb):
    w, b = wb
    return jnp.dot(x, w.T) + b


def _ff_j(x, pp):
    return jax.nn.softplus(_lin_j(jax.nn.softplus(_lin_j(x, pp[0])), pp[1]))


def _set2set(xx, ids_col, lp, lo, hi, t):
    w_ih, w_hh, b_ih, b_hh = lp
    q_star = jnp.zeros((64, 32), F32)
    h = jnp.zeros((64, 16), F32)
    cst = jnp.zeros((64, 16), F32)
    for _ in range(3):
        gates = jnp.dot(q_star, w_ih.T) + b_ih + jnp.dot(h, w_hh.T) + b_hh
        ig, fg, gg, og = jnp.split(gates, 4, axis=1)
        ig = jax.nn.sigmoid(ig)
        fg = jax.nn.sigmoid(fg)
        gg = jnp.tanh(gg)
        og = jax.nn.sigmoid(og)
        cst = fg * cst + ig * gg
        h = og * jnp.tanh(cst)
        q = h
        _, sexp, r = _s2s_pass(xx, ids_col, q, lo, hi, t)
        rg = (r / (sexp + 1e-16)).T
        q_star = jnp.concatenate([q, rg], axis=1)
    return q_star


def _t2(wb):
    w, b = wb
    return w.T, b.reshape(1, -1)


def kernel(x, edge_index, edge_attr, batch, params):
    p = params
    N = x.shape[0]
    E = edge_index.shape[1]
    unit = _NW * _CH * _LW
    epad = ((E + unit - 1) // unit) * unit
    npad = (N // 512 + 1) * 512
    trash = npad - 1
    re_rows = epad // _LW

    src = edge_index[0].astype(I32)
    dst = edge_index[1].astype(I32)
    padv = jnp.full((epad - E,), trash, I32)
    srcp = jnp.concatenate([src, padv]).reshape(re_rows, _LW)
    dstp = jnp.concatenate([dst, padv]).reshape(re_rows, _LW)
    src_col = src.reshape(E, 1)
    dst_col = dst.reshape(E, 1)

    starts = jnp.searchsorted(batch, jnp.arange(65, dtype=I32)).astype(I32)
    lo = starts[:64].reshape(1, 64)
    hi = starts[1:].reshape(1, 64)
    bcnt = jnp.maximum((starts[1:] - starts[:64]).astype(F32), 1.0).reshape(64, 1)

    e0 = _edge_init(edge_attr, *_t2(p['ffe0'][0]), *_t2(p['ffe0'][1]))
    v0 = _node_init(x, *_t2(p['embedding']), *_t2(p['ffv0'][0]), *_t2(p['ffv0'][1]))
    u0 = _ff_j(jnp.zeros((64, 2), F32), p['ffu0'])

    cnts = _sc_counts(dstp, npad)
    c0 = cnts[0]
    c1 = cnts[1]

    vres, eres, ures = v0, e0, u0
    for li, mk in enumerate(('meg1', 'meg2', 'meg3')):
        mp = p[mk]
        if li == 0:
            vff, uin, ew = vres, ures, None
        else:
            fv, fe, fu = p['ffv%d' % li], p['ffe%d' % li], p['ffu%d' % li]
            vff = _node_ff(vres, *_t2(fv[0]), *_t2(fv[1]))
            ew = (*_t2(fe[0]), *_t2(fe[1]))
            uin = _ff_j(ures, fu)
        tab = jnp.pad(vff, ((0, npad - N), (0, 0)))
        vs, vd = _sc_gather2(tab, srcp, dstp)
        cw_e = (*_t2(mp['e1']), *_t2(mp['e2']), *_t2(mp['e3']))
        ep, enext = _edge_conv(eres, vs, vd, src_col, lo, hi, uin, ew, cw_e, epad)
        sums = _sc_scatter(ep, dstp, npad)
        cw_v = (*_t2(mp['v1']), *_t2(mp['v2']), *_t2(mp['v3']))
        vnext, ue_s, uv_s = _node_conv(vres, vff, sums, c0, c1, uin, lo, hi, cw_v)
        ue = ue_s / bcnt
        uv = uv_s / bcnt
        uc = jnp.concatenate([ue, uv, uin], axis=1)
        upd = jax.nn.softplus(_lin_j(uc, mp['u1']))
        upd = jax.nn.softplus(_lin_j(upd, mp['u2']))
        upd = jax.nn.softplus(_lin_j(upd, mp['u3']))
        vres, eres, ures = vnext, enext, upd + ures

    xv = _linear16(vres, *_t2(p['node_linear']), _TN)
    xe = _linear16(eres, *_t2(p['edge_linear']), _TE)
    node_vec = _set2set(xv, None, p['lstm_node'], lo, hi, _TN)
    edge_vec = _set2set(xe, dst_col, p['lstm_edge'], lo, hi, _TE)
    fin = jnp.concatenate([node_vec, edge_vec, ures], axis=1)
    out = jax.nn.softplus(_lin_j(fin, p['fc1']))
    out = jax.nn.softplus(_lin_j(out, p['fc2']))
    return _lin_j(out, p['fc3'])
